# Initial kernel scaffold; baseline (speedup 1.0000x reference)
import functools

import jax
import jax.numpy as jnp
from jax import lax
from jax.experimental import pallas as pl
from jax.experimental.pallas import tpu as pltpu
from jax.experimental.pallas import tpu_sc as plsc

LANES = 16


def _make_histogram(n_edges: int, n_nodes: int, n_subcores: int):
    assert n_edges % (n_subcores * LANES) == 0
    assert n_nodes % LANES == 0
    chunk = n_edges // n_subcores
    mesh = plsc.VectorSubcoreMesh(
        core_axis_name="c", subcore_axis_name="s", num_cores=1
    )

    @functools.partial(
        pl.kernel,
        mesh=mesh,
        out_type=jax.ShapeDtypeStruct((n_nodes,), jnp.int32),
        scratch_types=[
            pltpu.VMEM((chunk,), jnp.int32),
            pltpu.VMEM((n_nodes,), jnp.int32),
            pltpu.VMEM_SHARED((n_nodes,), jnp.int32),
        ],
    )
    def hist_kernel(idx_hbm, out_hbm, idx_v, hist_v, shared):
        wid = lax.axis_index("s")
        zeros = jnp.zeros((LANES,), jnp.int32)
        ones = jnp.ones((LANES,), jnp.int32)

        def zero_body(i, _):
            hist_v[pl.ds(i * LANES, LANES)] = zeros
            return ()

        lax.fori_loop(0, n_nodes // LANES, zero_body, (), unroll=8)

        pltpu.sync_copy(idx_hbm.at[pl.ds(wid * chunk, chunk)], idx_v)

        def hist_body(i, _):
            v = idx_v[pl.ds(i * LANES, LANES)]
            plsc.addupdate_scatter(hist_v, [v], ones)
            return ()

        lax.fori_loop(0, chunk // LANES, hist_body, (), unroll=8)

        @pl.when(wid == 0)
        def _():
            pltpu.sync_copy(hist_v, shared)

        plsc.subcore_barrier()

        @pl.when(wid != 0)
        def _():
            pltpu.sync_copy(hist_v, shared, add=True)

        plsc.subcore_barrier()

        @pl.when(wid == 0)
        def _():
            pltpu.sync_copy(shared, out_hbm)

    return hist_kernel


def _ew_body(n_bits, cnt_ref, xs_ref, xp_ref, osum_ref, oprod_ref):
    cnt = cnt_ref[...]
    cf = cnt.astype(jnp.float32)
    osum_ref[...] = cf * xs_ref[...]

    base = xp_ref[...]
    acc = jnp.ones_like(base)
    for k in range(n_bits):
        bit = (cnt >> k) & 1
        acc = jnp.where(bit == 1, acc * base, acc)
        if k + 1 < n_bits:
            base = base * base
    oprod_ref[...] = acc


def _elementwise(counts, x_sum, x_prod, n_bits, block_rows):
    n, d = x_sum.shape
    grid = (n // block_rows,)
    bspec_c = pl.BlockSpec((block_rows, 1), lambda i: (i, 0))
    bspec_x = pl.BlockSpec((block_rows, d), lambda i: (i, 0))
    return pl.pallas_call(
        functools.partial(_ew_body, n_bits),
        grid=grid,
        in_specs=[bspec_c, bspec_x, bspec_x],
        out_specs=[bspec_x, bspec_x],
        out_shape=[
            jax.ShapeDtypeStruct((n, d), jnp.float32),
            jax.ShapeDtypeStruct((n, d), jnp.float32),
        ],
    )(counts, x_sum, x_prod)


@jax.jit
def kernel(edge_index, x_sum, x_prod):
    idx = edge_index[1].astype(jnp.int32)
    n_edges = idx.shape[0]
    n_nodes = x_sum.shape[0]

    counts = _make_histogram(n_edges, n_nodes, 16)(idx)
    counts_col = counts.reshape(n_nodes, 1)

    n_bits = max(1, n_edges.bit_length())
    block_rows = 1250 if n_nodes % 1250 == 0 else n_nodes
    out_sum, out_prod = _elementwise(counts_col, x_sum, x_prod, n_bits, block_rows)
    return (out_sum, out_prod)


# speedup vs baseline: 57.5334x; 57.5334x over previous
import functools

import jax
import jax.numpy as jnp
from jax import lax
from jax.experimental import pallas as pl
from jax.experimental.pallas import tpu as pltpu
from jax.experimental.pallas import tpu_sc as plsc

LANES = 16


def _make_histogram(n_edges: int, n_bins: int, n_subcores: int):
    assert n_edges % (n_subcores * LANES) == 0
    assert n_bins % (n_subcores * LANES) == 0
    chunk = n_edges // n_subcores
    slab = n_bins // n_subcores
    mesh = plsc.VectorSubcoreMesh(
        core_axis_name="c", subcore_axis_name="s", num_cores=1
    )

    @functools.partial(
        pl.kernel,
        mesh=mesh,
        compiler_params=pltpu.CompilerParams(needs_layout_passes=False),
        out_type=jax.ShapeDtypeStruct((n_bins,), jnp.int32),
        scratch_types=[
            pltpu.VMEM((chunk,), jnp.int32),
            pltpu.VMEM((n_bins,), jnp.int32),
            pltpu.VMEM((slab,), jnp.int32),
            pltpu.VMEM((slab,), jnp.int32),
            pltpu.VMEM_SHARED((n_subcores, n_bins), jnp.int32),
        ],
    )
    def hist_kernel(idx_hbm, out_hbm, idx_v, hist_v, tmp_v, acc_v, shared):
        wid = lax.axis_index("s")
        zeros = jnp.zeros((LANES,), jnp.int32)
        ones = jnp.ones((LANES,), jnp.int32)

        def zero_body(i, _):
            hist_v[pl.ds(i * LANES, LANES)] = zeros
            return ()

        lax.fori_loop(0, n_bins // LANES, zero_body, (), unroll=8)

        pltpu.sync_copy(idx_hbm.at[pl.ds(wid * chunk, chunk)], idx_v)

        def hist_body(i, _):
            v = idx_v[pl.ds(i * LANES, LANES)]
            plsc.addupdate_scatter(hist_v, [v], ones)
            return ()

        lax.fori_loop(0, chunk // LANES, hist_body, (), unroll=8)

        pltpu.sync_copy(hist_v, shared.at[wid])
        plsc.subcore_barrier()

        base = wid * slab
        pltpu.sync_copy(shared.at[0, pl.ds(base, slab)], acc_v)

        def red_body(h, _):
            pltpu.sync_copy(shared.at[h, pl.ds(base, slab)], tmp_v)
            for r in range(slab // LANES):
                sl = pl.ds(r * LANES, LANES)
                acc_v[sl] = acc_v[sl] + tmp_v[sl]
            return ()

        lax.fori_loop(1, n_subcores, red_body, ())

        pltpu.sync_copy(acc_v, out_hbm.at[pl.ds(base, slab)])

    return hist_kernel


def _ew_body(n_bits, cnt_ref, xs_ref, xp_ref, osum_ref, oprod_ref):
    cnt = cnt_ref[...]
    cf = cnt.astype(jnp.float32)
    osum_ref[...] = cf * xs_ref[...]

    base = xp_ref[...]
    acc = jnp.ones_like(base)
    for k in range(n_bits):
        bit = (cnt >> k) & 1
        acc = jnp.where(bit == 1, acc * base, acc)
        if k + 1 < n_bits:
            base = base * base
    oprod_ref[...] = acc


def _elementwise(counts, x_sum, x_prod, n_bits, block_rows):
    n, d = x_sum.shape
    grid = (n // block_rows,)
    bspec_c = pl.BlockSpec((block_rows, 1), lambda i: (i, 0))
    bspec_x = pl.BlockSpec((block_rows, d), lambda i: (i, 0))
    return pl.pallas_call(
        functools.partial(_ew_body, n_bits),
        grid=grid,
        in_specs=[bspec_c, bspec_x, bspec_x],
        out_specs=[bspec_x, bspec_x],
        out_shape=[
            jax.ShapeDtypeStruct((n, d), jnp.float32),
            jax.ShapeDtypeStruct((n, d), jnp.float32),
        ],
    )(counts, x_sum, x_prod)


@jax.jit
def kernel(edge_index, x_sum, x_prod):
    idx = edge_index[1].astype(jnp.int32)
    n_edges = idx.shape[0]
    n_nodes = x_sum.shape[0]

    n_sub = 16
    pad = n_sub * LANES
    n_bins = ((n_nodes + pad - 1) // pad) * pad
    counts = _make_histogram(n_edges, n_bins, n_sub)(idx)
    counts_col = counts[:n_nodes].reshape(n_nodes, 1)

    n_bits = max(1, n_edges.bit_length())
    block_rows = 2000 if n_nodes % 2000 == 0 else n_nodes
    out_sum, out_prod = _elementwise(counts_col, x_sum, x_prod, n_bits, block_rows)
    return (out_sum, out_prod)


# speedup vs baseline: 70.9676x; 1.2335x over previous
import functools

import jax
import jax.numpy as jnp
from jax import lax
from jax.experimental import pallas as pl
from jax.experimental.pallas import tpu as pltpu
from jax.experimental.pallas import tpu_sc as plsc

LANES = 16


def _make_histogram(n_edges: int, n_bins: int, n_cores: int, n_subcores: int):
    n_tiles = n_cores * n_subcores
    assert n_edges % (n_tiles * LANES) == 0
    assert n_bins % (n_subcores * LANES) == 0
    chunk = n_edges // n_tiles
    slab = n_bins // n_subcores
    mesh = plsc.VectorSubcoreMesh(
        core_axis_name="c", subcore_axis_name="s", num_cores=n_cores
    )

    @functools.partial(
        pl.kernel,
        mesh=mesh,
        compiler_params=pltpu.CompilerParams(needs_layout_passes=False),
        out_type=jax.ShapeDtypeStruct((n_cores, n_bins), jnp.int32),
        scratch_types=[
            pltpu.VMEM((chunk,), jnp.int32),
            pltpu.VMEM((n_bins,), jnp.int32),
            pltpu.VMEM((slab,), jnp.int32),
            pltpu.VMEM((slab,), jnp.int32),
            pltpu.VMEM_SHARED((n_subcores, n_bins), jnp.int32),
        ],
    )
    def hist_kernel(idx_hbm, out_hbm, idx_v, hist_v, tmp_v, acc_v, shared):
        cid = lax.axis_index("c")
        wid = lax.axis_index("s")
        tid = wid * n_cores + cid
        zeros = jnp.zeros((LANES,), jnp.int32)
        ones = jnp.ones((LANES,), jnp.int32)

        def zero_body(i, _):
            hist_v[pl.ds(i * LANES, LANES)] = zeros
            return ()

        lax.fori_loop(0, n_bins // LANES, zero_body, (), unroll=8)

        pltpu.sync_copy(idx_hbm.at[pl.ds(tid * chunk, chunk)], idx_v)

        def hist_body(i, _):
            v = idx_v[pl.ds(i * LANES, LANES)]
            plsc.addupdate_scatter(hist_v, [v], ones)
            return ()

        lax.fori_loop(0, chunk // LANES, hist_body, (), unroll=8)

        pltpu.sync_copy(hist_v, shared.at[wid])
        plsc.subcore_barrier()

        base = wid * slab
        pltpu.sync_copy(shared.at[0, pl.ds(base, slab)], acc_v)

        def red_body(h, _):
            pltpu.sync_copy(shared.at[h, pl.ds(base, slab)], tmp_v)
            for r in range(slab // LANES):
                sl = pl.ds(r * LANES, LANES)
                acc_v[sl] = acc_v[sl] + tmp_v[sl]
            return ()

        lax.fori_loop(1, n_subcores, red_body, ())

        pltpu.sync_copy(acc_v, out_hbm.at[cid, pl.ds(base, slab)])

    return hist_kernel


_LOW_BITS = 8


def _ew_body(n_bits, c0_ref, c1_ref, xs_ref, xp_ref, osum_ref, oprod_ref):
    cnt = c0_ref[...] + c1_ref[...]
    cf = cnt.astype(jnp.float32)
    osum_ref[...] = cf * xs_ref[...]

    x = xp_ref[...]
    lo = cnt & ((1 << _LOW_BITS) - 1)
    base = x
    acc = jnp.ones_like(base)
    for k in range(_LOW_BITS):
        bit = (lo >> k) & 1
        acc = jnp.where(bit == 1, acc * base, acc)
        if k + 1 < _LOW_BITS:
            base = base * base
    if n_bits > _LOW_BITS:
        hi = cnt >> _LOW_BITS
        hi_f = hi.astype(jnp.float32) * float(1 << _LOW_BITS)
        hi_pow = jnp.exp(hi_f * jnp.log(x))
        acc = jnp.where(hi > 0, acc * hi_pow, acc)
    oprod_ref[...] = acc


def _elementwise(c0, c1, x_sum, x_prod, n_bits, block_rows):
    n, d = x_sum.shape
    grid = (n // block_rows,)
    bspec_c = pl.BlockSpec((block_rows, 1), lambda i: (i, 0))
    bspec_x = pl.BlockSpec((block_rows, d), lambda i: (i, 0))
    return pl.pallas_call(
        functools.partial(_ew_body, n_bits),
        grid=grid,
        in_specs=[bspec_c, bspec_c, bspec_x, bspec_x],
        out_specs=[bspec_x, bspec_x],
        out_shape=[
            jax.ShapeDtypeStruct((n, d), jnp.float32),
            jax.ShapeDtypeStruct((n, d), jnp.float32),
        ],
    )(c0, c1, x_sum, x_prod)


@jax.jit
def kernel(edge_index, x_sum, x_prod):
    idx = edge_index[1].astype(jnp.int32)
    n_edges = idx.shape[0]
    n_nodes = x_sum.shape[0]

    n_cores = 2
    n_sub = 16
    pad = n_sub * LANES
    n_bins = ((n_nodes + pad - 1) // pad) * pad
    partials = _make_histogram(n_edges, n_bins, n_cores, n_sub)(idx)
    c0 = partials[0, :n_nodes].reshape(n_nodes, 1)
    c1 = partials[1, :n_nodes].reshape(n_nodes, 1)

    n_bits = max(1, n_edges.bit_length())
    block_rows = 2000 if n_nodes % 2000 == 0 else n_nodes
    out_sum, out_prod = _elementwise(c0, c1, x_sum, x_prod, n_bits, block_rows)
    return (out_sum, out_prod)
